# baseline (speedup 1.0000x reference)
import functools

import jax
import jax.numpy as jnp
from jax import lax
from jax.experimental import pallas as pl
from jax.experimental.pallas import tpu as pltpu
from jax.experimental.pallas import tpu_sc as plsc

NUM_MODELS = 8
NUM_BOXES = 100000
DIM = 128
BATCH = 16384

NW = 32
ROWS_PER_W = NUM_MODELS * BATCH // NW
R = 64
NCHUNK = ROWS_PER_W // R
NSLOT = 6


def _body(z_hbm, ld_hbm, ids_hbm, out_hbm, idx_v, ldbuf, obuf,
          sem_z, sem_l, sem_o):
    c = lax.axis_index("c")
    s = lax.axis_index("s")
    wid = s * 2 + c
    m = wid // 4
    bwin = wid % 4

    pltpu.sync_copy(ids_hbm.at[bwin], idx_v)
    moff = m * NUM_BOXES

    @plsc.parallel_loop(0, NCHUNK, unroll=2)
    def _add_off(g):
        for j in range(R // 16):
            sl = pl.ds(j * 16, 16)
            idx_v[g, sl] = idx_v[g, sl] + moff

    out_base = wid * ROWS_PER_W

    def gather_start(g, b):
        pltpu.async_copy(z_hbm.at[idx_v.at[g]], obuf.at[b], sem_z.at[b])
        pltpu.async_copy(ld_hbm.at[idx_v.at[g]], ldbuf.at[b], sem_l.at[b])

    def gather_wait(g, b):
        pltpu.make_async_copy(z_hbm.at[idx_v.at[g]], obuf.at[b],
                              sem_z.at[b]).wait()
        pltpu.make_async_copy(ld_hbm.at[idx_v.at[g]], ldbuf.at[b],
                              sem_l.at[b]).wait()

    def store_start(g, b):
        pltpu.async_copy(obuf.at[b], out_hbm.at[pl.ds(out_base + g * R, R)],
                         sem_o.at[b])

    def store_wait(g, b):
        pltpu.make_async_copy(obuf.at[b],
                              out_hbm.at[pl.ds(out_base + g * R, R)],
                              sem_o.at[b]).wait()

    def compute(b):
        @plsc.parallel_loop(0, R, unroll=4)
        def _cmp(r):
            for j in range(DIM // 16):
                sl = pl.ds(j * 16, 16)
                obuf[b, r, sl] = obuf[b, r, sl] + jnp.exp(ldbuf[b, r, sl])

    def sw(g):
        for b in range(NSLOT):
            @pl.when(g % NSLOT == b)
            def _():
                store_wait(g, b)

    for g in range(NSLOT - 1):
        gather_start(g, g % NSLOT)

    def step(g, carry):
        pre = g + NSLOT - 1

        for b in range(NSLOT):
            @pl.when(g % NSLOT == b)
            def _():
                gather_wait(g, b)

        @pl.when(pre < NCHUNK)
        def _():
            @pl.when(g >= 1)
            def _():
                sw(g - 1)
            for b in range(NSLOT):
                @pl.when(pre % NSLOT == b)
                def _():
                    gather_start(pre, b)

        for b in range(NSLOT):
            @pl.when(g % NSLOT == b)
            def _():
                compute(b)
                store_start(g, b)
        return carry

    lax.fori_loop(0, NCHUNK, step, 0)
    for g in range(NCHUNK - NSLOT, NCHUNK):
        store_wait(g, g % NSLOT)


@jax.jit
def _sc_lookup(zf, lf, ids3):
    mesh = plsc.VectorSubcoreMesh(core_axis_name="c", subcore_axis_name="s")
    fn = pl.kernel(
        _body,
        mesh=mesh,
        out_type=jax.ShapeDtypeStruct((NUM_MODELS * BATCH, DIM), jnp.float32),
        scratch_types=[
            pltpu.VMEM((NCHUNK, R), jnp.int32),
            pltpu.VMEM((NSLOT, R, DIM), jnp.float32),
            pltpu.VMEM((NSLOT, R, DIM), jnp.float32),
            pltpu.SemaphoreType.DMA((NSLOT,)),
            pltpu.SemaphoreType.DMA((NSLOT,)),
            pltpu.SemaphoreType.DMA((NSLOT,)),
        ],
    )
    return fn(zf, lf, ids3)


def kernel(z, logdelta, ids):
    zf = z.reshape(NUM_MODELS * NUM_BOXES, DIM)
    lf = logdelta.reshape(NUM_MODELS * NUM_BOXES, DIM)
    ids3 = ids.astype(jnp.int32).reshape(NW // NUM_MODELS, NCHUNK, R)
    out = _sc_lookup(zf, lf, ids3)
    return out.reshape(NUM_MODELS, BATCH, DIM)


# speedup vs baseline: 1.0289x; 1.0289x over previous
import functools

import jax
import jax.numpy as jnp
from jax import lax
from jax.experimental import pallas as pl
from jax.experimental.pallas import tpu as pltpu
from jax.experimental.pallas import tpu_sc as plsc

NUM_MODELS = 8
NUM_BOXES = 100000
DIM = 128
BATCH = 16384

NW = 32
ROWS_PER_W = NUM_MODELS * BATCH // NW
R = 64
NCHUNK = ROWS_PER_W // R
NSLOT = 3


def _body(z_hbm, ld_hbm, ids_hbm, out_hbm, idx_v, ldbuf, obuf, sbuf,
          sem_z, sem_l, sem_c, sem_h):
    c = lax.axis_index("c")
    s = lax.axis_index("s")
    wid = s * 2 + c
    m = wid // 4
    bwin = wid % 4

    pltpu.sync_copy(ids_hbm.at[bwin], idx_v)
    moff = m * NUM_BOXES

    @plsc.parallel_loop(0, NCHUNK, unroll=2)
    def _add_off(g):
        for j in range(R // 16):
            sl = pl.ds(j * 16, 16)
            idx_v[g, sl] = idx_v[g, sl] + moff

    out_base = wid * ROWS_PER_W

    def gather_start(g, b):
        pltpu.async_copy(z_hbm.at[idx_v.at[g]], obuf.at[b], sem_z.at[b])
        pltpu.async_copy(ld_hbm.at[idx_v.at[g]], ldbuf.at[b], sem_l.at[b])

    def gather_wait(g, b):
        pltpu.make_async_copy(z_hbm.at[idx_v.at[g]], obuf.at[b],
                              sem_z.at[b]).wait()
        pltpu.make_async_copy(ld_hbm.at[idx_v.at[g]], ldbuf.at[b],
                              sem_l.at[b]).wait()

    def cstart(g, b):
        del g
        pltpu.async_copy(obuf.at[b], sbuf.at[s, b], sem_c.at[b])

    def cwait(g, b):
        del g
        pltpu.make_async_copy(obuf.at[b], sbuf.at[s, b], sem_c.at[b]).wait()

    def hstart(g, b):
        pltpu.async_copy(sbuf.at[s, b],
                         out_hbm.at[pl.ds(out_base + g * R, R)], sem_h.at[b])

    def hwait(g, b):
        pltpu.make_async_copy(sbuf.at[s, b],
                              out_hbm.at[pl.ds(out_base + g * R, R)],
                              sem_h.at[b]).wait()

    def compute(b):
        @plsc.parallel_loop(0, R, unroll=4)
        def _cmp(r):
            for j in range(DIM // 16):
                sl = pl.ds(j * 16, 16)
                obuf[b, r, sl] = obuf[b, r, sl] + jnp.exp(ldbuf[b, r, sl])

    def dispatch(g, fn):
        for b in range(NSLOT):
            @pl.when(g % NSLOT == b)
            def _():
                fn(g, b)

    for g in range(NSLOT - 1):
        gather_start(g, g % NSLOT)

    def step(g, carry):
        pre = g + NSLOT - 1

        dispatch(g, gather_wait)

        @pl.when(g >= 1)
        def _():
            dispatch(g - 1, cwait)
            dispatch(g - 1, hstart)

        @pl.when(pre < NCHUNK)
        def _():
            dispatch(pre, gather_start)

        @pl.when(g >= NSLOT)
        def _():
            dispatch(g - NSLOT, hwait)

        dispatch(g, lambda gg, b: (compute(b), cstart(gg, b)))
        return carry

    lax.fori_loop(0, NCHUNK, step, 0)
    cwait(NCHUNK - 1, (NCHUNK - 1) % NSLOT)
    hstart(NCHUNK - 1, (NCHUNK - 1) % NSLOT)
    for g in range(NCHUNK - NSLOT, NCHUNK):
        hwait(g, g % NSLOT)


@jax.jit
def _sc_lookup(zf, lf, ids3):
    mesh = plsc.VectorSubcoreMesh(core_axis_name="c", subcore_axis_name="s")
    fn = pl.kernel(
        _body,
        mesh=mesh,
        out_type=jax.ShapeDtypeStruct((NUM_MODELS * BATCH, DIM), jnp.float32),
        scratch_types=[
            pltpu.VMEM((NCHUNK, R), jnp.int32),
            pltpu.VMEM((NSLOT, R, DIM), jnp.float32),
            pltpu.VMEM((NSLOT, R, DIM), jnp.float32),
            pltpu.VMEM_SHARED((16, NSLOT, R, DIM), jnp.float32),
            pltpu.SemaphoreType.DMA((NSLOT,)),
            pltpu.SemaphoreType.DMA((NSLOT,)),
            pltpu.SemaphoreType.DMA((NSLOT,)),
            pltpu.SemaphoreType.DMA((NSLOT,)),
        ],
    )
    return fn(zf, lf, ids3)


def kernel(z, logdelta, ids):
    zf = z.reshape(NUM_MODELS * NUM_BOXES, DIM)
    lf = logdelta.reshape(NUM_MODELS * NUM_BOXES, DIM)
    ids3 = ids.astype(jnp.int32).reshape(NW // NUM_MODELS, NCHUNK, R)
    out = _sc_lookup(zf, lf, ids3)
    return out.reshape(NUM_MODELS, BATCH, DIM)


# speedup vs baseline: 1.0331x; 1.0041x over previous
import functools

import jax
import jax.numpy as jnp
from jax import lax
from jax.experimental import pallas as pl
from jax.experimental.pallas import tpu as pltpu
from jax.experimental.pallas import tpu_sc as plsc

NUM_MODELS = 8
NUM_BOXES = 100000
DIM = 128
BATCH = 16384

NW = 32
ROWS_PER_W = NUM_MODELS * BATCH // NW
R = 64
NCHUNK = ROWS_PER_W // R
NSLOT = 3


def _body(z_hbm, ld_hbm, ids_hbm, out_hbm, idx_v, ldbuf, obuf, sbuf,
          sem_z, sem_l, sem_c, sem_h):
    c = lax.axis_index("c")
    s = lax.axis_index("s")
    wid = s * 2 + c

    pltpu.sync_copy(ids_hbm.at[wid, pl.ds(0, 8)], idx_v.at[pl.ds(0, 8)])

    out_base = wid * ROWS_PER_W

    def gather_start(g, b):
        pltpu.async_copy(z_hbm.at[idx_v.at[g]], obuf.at[b], sem_z.at[b])
        pltpu.async_copy(ld_hbm.at[idx_v.at[g]], ldbuf.at[b], sem_l.at[b])

    def gather_wait(g, b):
        pltpu.make_async_copy(z_hbm.at[idx_v.at[g]], obuf.at[b],
                              sem_z.at[b]).wait()
        pltpu.make_async_copy(ld_hbm.at[idx_v.at[g]], ldbuf.at[b],
                              sem_l.at[b]).wait()

    def cstart(g, b):
        del g
        pltpu.async_copy(obuf.at[b], sbuf.at[s, b], sem_c.at[b])

    def cwait(g, b):
        del g
        pltpu.make_async_copy(obuf.at[b], sbuf.at[s, b], sem_c.at[b]).wait()

    def hstart(g, b):
        pltpu.async_copy(sbuf.at[s, b],
                         out_hbm.at[pl.ds(out_base + g * R, R)], sem_h.at[b])

    def hwait(g, b):
        pltpu.make_async_copy(sbuf.at[s, b],
                              out_hbm.at[pl.ds(out_base + g * R, R)],
                              sem_h.at[b]).wait()

    def compute(b):
        @plsc.parallel_loop(0, R, unroll=4)
        def _cmp(r):
            for j in range(DIM // 16):
                sl = pl.ds(j * 16, 16)
                obuf[b, r, sl] = obuf[b, r, sl] + jnp.exp(ldbuf[b, r, sl])

    def dispatch(g, fn):
        for b in range(NSLOT):
            @pl.when(g % NSLOT == b)
            def _():
                fn(g, b)

    for g in range(NSLOT - 1):
        gather_start(g, g % NSLOT)
    pltpu.sync_copy(ids_hbm.at[wid, pl.ds(8, NCHUNK - 8)],
                    idx_v.at[pl.ds(8, NCHUNK - 8)])

    def step(g, carry):
        pre = g + NSLOT - 1

        dispatch(g, gather_wait)

        @pl.when(g >= 1)
        def _():
            dispatch(g - 1, cwait)
            dispatch(g - 1, hstart)

        @pl.when(pre < NCHUNK)
        def _():
            dispatch(pre, gather_start)

        @pl.when(g >= NSLOT)
        def _():
            dispatch(g - NSLOT, hwait)

        dispatch(g, lambda gg, b: (compute(b), cstart(gg, b)))
        return carry

    lax.fori_loop(0, NCHUNK, step, 0)
    cwait(NCHUNK - 1, (NCHUNK - 1) % NSLOT)
    hstart(NCHUNK - 1, (NCHUNK - 1) % NSLOT)
    for g in range(NCHUNK - NSLOT, NCHUNK):
        hwait(g, g % NSLOT)


@jax.jit
def _sc_lookup(zf, lf, ids3):
    mesh = plsc.VectorSubcoreMesh(core_axis_name="c", subcore_axis_name="s")
    fn = pl.kernel(
        _body,
        mesh=mesh,
        out_type=jax.ShapeDtypeStruct((NUM_MODELS * BATCH, DIM), jnp.float32),
        scratch_types=[
            pltpu.VMEM((NCHUNK, R), jnp.int32),
            pltpu.VMEM((NSLOT, R, DIM), jnp.float32),
            pltpu.VMEM((NSLOT, R, DIM), jnp.float32),
            pltpu.VMEM_SHARED((16, NSLOT, R, DIM), jnp.float32),
            pltpu.SemaphoreType.DMA((NSLOT,)),
            pltpu.SemaphoreType.DMA((NSLOT,)),
            pltpu.SemaphoreType.DMA((NSLOT,)),
            pltpu.SemaphoreType.DMA((NSLOT,)),
        ],
    )
    return fn(zf, lf, ids3)


def kernel(z, logdelta, ids):
    zf = z.reshape(NUM_MODELS * NUM_BOXES, DIM)
    lf = logdelta.reshape(NUM_MODELS * NUM_BOXES, DIM)
    offs = (jnp.arange(NUM_MODELS, dtype=jnp.int32) * NUM_BOXES)[:, None]
    ids4 = (ids.astype(jnp.int32)[None, :] + offs).reshape(NW, NCHUNK, R)
    out = _sc_lookup(zf, lf, ids4)
    return out.reshape(NUM_MODELS, BATCH, DIM)
